# Initial kernel scaffold; baseline (speedup 1.0000x reference)
import jax
import jax.numpy as jnp
from jax.experimental import pallas as pl


def kernel(edge_attr, x, global_attr, W, b, edge_index):
    raise NotImplementedError("write your pallas kernel here")


# speedup vs baseline: 3.2537x; 3.2537x over previous
import functools

import jax
import jax.numpy as jnp
from jax import lax
from jax.experimental import pallas as pl
from jax.experimental.pallas import tpu as pltpu
from jax.experimental.pallas import tpu_sc as plsc

N_NODES = 10000
N_EDGES = 320000
D_FEAT = 128
D_EDGE = 16
D_OUT = 128

NC = 2
NS = 16
NW = NC * NS

E2 = 2 * N_EDGES
RW = E2 // NW
CHUNK = 400
GB = 80
NCHUNK = RW // CHUNK


def _table_body(x_ref, w_ref, g_ref, wg_ref, b_ref, t_ref):
    c = jnp.dot(g_ref[...], wg_ref[...], preferred_element_type=jnp.float32)
    c = (c + b_ref[...]) * 0.5
    t_ref[0] = (
        jnp.dot(x_ref[...], w_ref[0], preferred_element_type=jnp.float32) + c
    )


def _build_table(x, wrs, g2d, wg, b2d):
    t3 = pl.pallas_call(
        _table_body,
        grid=(2,),
        in_specs=[
            pl.BlockSpec((N_NODES, D_FEAT), lambda j: (0, 0)),
            pl.BlockSpec((1, D_FEAT, D_OUT), lambda j: (j, 0, 0)),
            pl.BlockSpec((1, D_FEAT), lambda j: (0, 0)),
            pl.BlockSpec((D_FEAT, D_OUT), lambda j: (0, 0)),
            pl.BlockSpec((1, D_OUT), lambda j: (0, 0)),
        ],
        out_specs=pl.BlockSpec((1, N_NODES, D_OUT), lambda j: (j, 0, 0)),
        out_shape=jax.ShapeDtypeStruct((2, N_NODES, D_OUT), jnp.float32),
    )(x, wrs, g2d, wg, b2d)
    return t3.reshape(2 * N_NODES, D_OUT)


def _gather_body(t_hbm, idx_hbm, out_hbm, idx_v, rows_v, sem):
    wid = lax.axis_index("s") * NC + lax.axis_index("c")
    base = wid * RW

    def chunk_body(ci, carry):
        off = base + ci * CHUNK
        pltpu.sync_copy(idx_hbm.at[pl.ds(off, CHUNK)], idx_v)
        copies = [
            pltpu.async_copy(
                t_hbm.at[idx_v.at[pl.ds(g * GB, GB)]],
                rows_v.at[pl.ds(g * GB, GB)],
                sem,
            )
            for g in range(CHUNK // GB)
        ]
        for c in copies:
            c.wait()
        pltpu.sync_copy(rows_v, out_hbm.at[pl.ds(off, CHUNK)])
        return carry

    lax.fori_loop(0, NCHUNK, chunk_body, 0)


@functools.cache
def _make_gather_rows():
    return pl.kernel(
        _gather_body,
        out_type=jax.ShapeDtypeStruct((E2, D_OUT), jnp.float32),
        mesh=plsc.VectorSubcoreMesh(
            core_axis_name="c",
            subcore_axis_name="s",
            num_cores=NC,
            num_subcores=NS,
        ),
        scratch_types=[
            pltpu.VMEM((CHUNK,), jnp.int32),
            pltpu.VMEM((CHUNK, D_OUT), jnp.float32),
            pltpu.SemaphoreType.DMA,
        ],
    )


def _gather_rows(table, idx2):
    return _make_gather_rows()(table, idx2)


BE = 2000
NB = N_EDGES // BE


def _fuse_body(ea_ref, we_ref, gr_ref, gs_ref, o_ref):
    acc = jnp.dot(ea_ref[...], we_ref[...], preferred_element_type=jnp.float32)
    o_ref[...] = jnp.maximum(acc + gr_ref[...] + gs_ref[...], 0.0)


def _fuse(edge_attr, we, g_rows):
    return pl.pallas_call(
        _fuse_body,
        grid=(NB,),
        in_specs=[
            pl.BlockSpec((BE, D_EDGE), lambda i: (i, 0)),
            pl.BlockSpec((D_EDGE, D_OUT), lambda i: (0, 0)),
            pl.BlockSpec((BE, D_OUT), lambda i: (i, 0)),
            pl.BlockSpec((BE, D_OUT), lambda i: (i + NB, 0)),
        ],
        out_specs=pl.BlockSpec((BE, D_OUT), lambda i: (i, 0)),
        out_shape=jax.ShapeDtypeStruct((N_EDGES, D_OUT), jnp.float32),
    )(edge_attr, we, g_rows, g_rows)


def kernel(edge_attr, x, global_attr, W, b, edge_index):
    we = W[:D_EDGE]
    wrs = W[D_EDGE:D_EDGE + 2 * D_FEAT].reshape(2, D_FEAT, D_OUT)
    wg = W[D_EDGE + 2 * D_FEAT:]
    g2d = global_attr.reshape(1, D_FEAT)
    b2d = b.reshape(1, D_OUT)

    table = _build_table(x, wrs, g2d, wg, b2d)

    idx2 = jnp.concatenate(
        [edge_index[0], edge_index[1] + N_NODES]
    ).astype(jnp.int32)
    g_rows = _gather_rows(table, idx2)

    return _fuse(edge_attr, we, g_rows)
